# baseline (speedup 1.0000x reference)
import functools

import jax
import jax.numpy as jnp
from jax import lax
from jax.experimental import pallas as pl
from jax.experimental.pallas import tpu as pltpu
from jax.experimental.pallas import tpu_sc as plsc

_L = 16
_IR = 128


def _build(N, V, D, NW, num_cores):
    C = 512
    CR = C // _IR
    rows_w = N // NW
    n_chunks = rows_w // C
    n_pairs = n_chunks // 2
    irows_w = rows_w // _IR

    mesh = plsc.VectorSubcoreMesh(
        core_axis_name="c", subcore_axis_name="s", num_cores=num_cores
    )

    @functools.partial(
        pl.kernel,
        mesh=mesh,
        compiler_params=pltpu.CompilerParams(use_tc_tiling_on_sc=False),
        out_type=jax.ShapeDtypeStruct((N, D), jnp.float32),
        scratch_types=[
            pltpu.VMEM((2, CR, _IR), jnp.int32),
            pltpu.VMEM((2, CR, _IR), jnp.int32),
            pltpu.VMEM((2, C, D), jnp.float32),
            pltpu.SemaphoreType.DMA,
            pltpu.SemaphoreType.DMA,
            pltpu.SemaphoreType.DMA,
            pltpu.SemaphoreType.DMA,
        ],
    )
    def k(xf_hbm, table_hbm, out_hbm, xbuf, idxbuf, rowbuf, g0, g1, o0, o1):
        nc = lax.axis_size("c")
        wid = lax.axis_index("s") * nc + lax.axis_index("c")
        irow0 = wid * irows_w
        gsem = (g0, g1)
        osem = (o0, o1)

        def fire_gathers(slot, g):
            r0 = irow0 + g * CR
            pltpu.sync_copy(xf_hbm.at[pl.ds(r0, CR)], xbuf.at[slot])
            for r in range(CR):
                for v in range(_IR // _L):
                    s = pl.ds(v * _L, _L)
                    idxbuf[slot, r, s] = -xbuf[slot, r, s] - 1
            for j in range(CR):
                pltpu.async_copy(
                    table_hbm.at[idxbuf.at[slot].at[j]],
                    rowbuf.at[slot].at[pl.ds(j * _IR, _IR)],
                    gsem[slot],
                )

        def drain_gathers(slot):
            for j in range(CR):
                pltpu.make_async_copy(
                    table_hbm.at[idxbuf.at[slot].at[j]],
                    rowbuf.at[slot].at[pl.ds(j * _IR, _IR)],
                    gsem[slot],
                ).wait()

        def out_copy(slot, g):
            r0 = irow0 + g * CR
            return pltpu.make_async_copy(
                rowbuf.at[slot], out_hbm.at[pl.ds(r0 * _IR, C)], osem[slot]
            )

        def body(p, carry):
            gc0 = 2 * p

            @pl.when(p >= 1)
            def _finish_prev_slot1():
                drain_gathers(1)
                out_copy(1, gc0 - 1).start()
                out_copy(0, gc0 - 2).wait()

            fire_gathers(0, gc0)

            @pl.when(p >= 1)
            def _free_slot1():
                out_copy(1, gc0 - 1).wait()

            fire_gathers(1, gc0 + 1)
            drain_gathers(0)
            out_copy(0, gc0).start()
            return carry

        lax.fori_loop(0, n_pairs, body, 0)
        drain_gathers(1)
        out_copy(1, n_chunks - 1).start()
        out_copy(0, n_chunks - 2).wait()
        out_copy(1, n_chunks - 1).wait()

    return k


def kernel(x, table):
    B, H = x.shape
    V, D = table.shape
    info = plsc.get_sparse_core_info()
    NS = info.num_subcores
    Hh = H // 2
    Nh = B * Hh
    k = _build(Nh, V, D, NS, num_cores=1)
    halves = []
    for h0 in (0, Hh):
        xf = x[:, h0:h0 + Hh].reshape(Nh // _IR, _IR).astype(jnp.int32)
        halves.append(k(xf, table).reshape(B, Hh, D))
    return jnp.concatenate(halves, axis=1)


# speedup vs baseline: 1.0127x; 1.0127x over previous
import functools

import jax
import jax.numpy as jnp
from jax import lax
from jax.experimental import pallas as pl
from jax.experimental.pallas import tpu as pltpu
from jax.experimental.pallas import tpu_sc as plsc

_L = 16
_IR = 128


def _build(N, V, D, NW, num_cores):
    C = 512
    CR = C // _IR
    rows_w = N // NW
    n_chunks = rows_w // C
    n_pairs = n_chunks // 2
    irows_w = rows_w // _IR

    mesh = plsc.VectorSubcoreMesh(
        core_axis_name="c", subcore_axis_name="s", num_cores=num_cores
    )

    @functools.partial(
        pl.kernel,
        mesh=mesh,
        compiler_params=pltpu.CompilerParams(use_tc_tiling_on_sc=False),
        out_type=jax.ShapeDtypeStruct((N, D), jnp.float32),
        scratch_types=[
            pltpu.VMEM((2, CR, _IR), jnp.int32),
            pltpu.VMEM((2, CR, _IR), jnp.int32),
            pltpu.VMEM((2, C, D), jnp.float32),
            pltpu.SemaphoreType.DMA,
            pltpu.SemaphoreType.DMA,
            pltpu.SemaphoreType.DMA,
            pltpu.SemaphoreType.DMA,
        ],
    )
    def k(xf_hbm, table_hbm, out_hbm, xbuf, idxbuf, rowbuf, g0, g1, o0, o1):
        nc = lax.axis_size("c")
        wid = lax.axis_index("s") * nc + lax.axis_index("c")
        irow0 = wid * irows_w
        gsem = (g0, g1)
        osem = (o0, o1)

        def fire_gathers(slot, g):
            r0 = irow0 + g * CR
            pltpu.sync_copy(xf_hbm.at[pl.ds(r0, CR)], xbuf.at[slot])
            for r in range(CR):
                for v in range(_IR // _L):
                    s = pl.ds(v * _L, _L)
                    idxbuf[slot, r, s] = -xbuf[slot, r, s] - 1
            for j in range(CR):
                pltpu.async_copy(
                    table_hbm.at[idxbuf.at[slot].at[j]],
                    rowbuf.at[slot].at[pl.ds(j * _IR, _IR)],
                    gsem[slot],
                )

        def drain_gathers(slot):
            for j in range(CR):
                pltpu.make_async_copy(
                    table_hbm.at[idxbuf.at[slot].at[j]],
                    rowbuf.at[slot].at[pl.ds(j * _IR, _IR)],
                    gsem[slot],
                ).wait()

        def out_copy(slot, g):
            r0 = irow0 + g * CR
            return pltpu.make_async_copy(
                rowbuf.at[slot], out_hbm.at[pl.ds(r0 * _IR, C)], osem[slot]
            )

        def body(p, carry):
            gc0 = 2 * p

            @pl.when(p >= 1)
            def _finish_prev_slot1():
                drain_gathers(1)
                out_copy(1, gc0 - 1).start()
                out_copy(0, gc0 - 2).wait()

            fire_gathers(0, gc0)

            @pl.when(p >= 1)
            def _free_slot1():
                out_copy(1, gc0 - 1).wait()

            fire_gathers(1, gc0 + 1)
            drain_gathers(0)
            out_copy(0, gc0).start()
            return carry

        lax.fori_loop(0, n_pairs, body, 0)
        drain_gathers(1)
        out_copy(1, n_chunks - 1).start()
        out_copy(0, n_chunks - 2).wait()
        out_copy(1, n_chunks - 1).wait()

    return k


def kernel(x, table):
    B, H = x.shape
    V, D = table.shape
    info = plsc.get_sparse_core_info()
    NS = info.num_subcores
    Hh = H // 2
    Nh = B * Hh
    k = _build(Nh, V, D, 2 * NS, num_cores=2)
    halves = []
    for h0 in (0, Hh):
        xf = x[:, h0:h0 + Hh].reshape(Nh // _IR, _IR).astype(jnp.int32)
        halves.append(k(xf, table).reshape(B, Hh, D))
    return jnp.concatenate(halves, axis=1)


# speedup vs baseline: 3.9537x; 3.9041x over previous
import functools

import jax
import jax.numpy as jnp
from jax import lax
from jax.experimental import pallas as pl
from jax.experimental.pallas import tpu as pltpu
from jax.experimental.pallas import tpu_sc as plsc

_L = 16
_G = 128
_XB = 32


def _build(B, H, V, D, NW):
    n_groups = (B // _G) * H
    gpw = n_groups // NW
    n_pairs = gpw // 2
    DT, DS = D // 8, 8

    mesh = plsc.VectorSubcoreMesh(core_axis_name="c", subcore_axis_name="s")

    @functools.partial(
        pl.kernel,
        mesh=mesh,
        compiler_params=pltpu.CompilerParams(
            use_tc_tiling_on_sc=False, needs_layout_passes=False
        ),
        out_type=jax.ShapeDtypeStruct((H, DT, B // _G, DS, _G), jnp.float32),
        scratch_types=[
            pltpu.VMEM((_XB, _G), jnp.int32),
            pltpu.VMEM((2, _G), jnp.int32),
            pltpu.VMEM((2, _G, D), jnp.float32),
            pltpu.VMEM((2, DT, DS, _G), jnp.float32),
            pltpu.SemaphoreType.DMA,
            pltpu.SemaphoreType.DMA,
            pltpu.SemaphoreType.DMA,
            pltpu.SemaphoreType.DMA,
        ],
    )
    def k(xg_hbm, table_hbm, out_hbm, xbuf, idxbuf, rowbuf, tbuf, g0, g1, w0, w1):
        nc = lax.axis_size("c")
        wid = lax.axis_index("s") * nc + lax.axis_index("c")
        gbase = wid * gpw
        gsem = (g0, g1)
        wsem = (w0, w1)
        bidx = [jax.lax.iota(jnp.int32, _L) + bv * _L for bv in range(_G // _L)]

        def make_idx(slot, r):
            for v in range(_G // _L):
                s = pl.ds(v * _L, _L)
                idxbuf[slot, s] = -xbuf[r, s] - 1

        def fire_gather(slot):
            pltpu.async_copy(
                table_hbm.at[idxbuf.at[slot]], rowbuf.at[slot], gsem[slot]
            )

        def wait_gather(slot):
            pltpu.make_async_copy(
                table_hbm.at[idxbuf.at[slot]], rowbuf.at[slot], gsem[slot]
            ).wait()

        def transpose(slot):
            def tb(d, carry):
                dt = lax.div(d, DS)
                ds_ = lax.rem(d, DS)
                idv = jnp.full((_L,), d, dtype=jnp.int32)
                for bv in range(_G // _L):
                    vec = plsc.load_gather(rowbuf.at[slot], [bidx[bv], idv])
                    tbuf[slot, dt, ds_, pl.ds(bv * _L, _L)] = vec
                return carry

            lax.fori_loop(0, D, tb, 0)

        def write_desc(slot, g):
            h = lax.div(g, B // _G)
            bt = lax.rem(g, B // _G)
            return pltpu.make_async_copy(
                tbuf.at[slot], out_hbm.at[h, :, bt], wsem[slot]
            )

        def body(p, carry):
            li0 = 2 * p
            gg0 = gbase + li0

            @pl.when(lax.rem(li0, _XB) == 0)
            def _load_x():
                pltpu.sync_copy(xg_hbm.at[pl.ds(gg0, _XB)], xbuf)

            r0 = lax.rem(li0, _XB)
            make_idx(0, r0)
            fire_gather(0)

            @pl.when(p >= 1)
            def _finish_prev_slot1():
                wait_gather(1)

                @pl.when(p >= 2)
                def _w1():
                    write_desc(1, gg0 - 3).wait()

                transpose(1)
                write_desc(1, gg0 - 1).start()

            make_idx(1, r0 + 1)
            fire_gather(1)

            wait_gather(0)

            @pl.when(p >= 1)
            def _w0():
                write_desc(0, gg0 - 2).wait()

            transpose(0)
            write_desc(0, gg0).start()
            return carry

        lax.fori_loop(0, n_pairs, body, 0)
        glast = gbase + gpw - 1
        wait_gather(1)
        write_desc(1, glast - 2).wait()
        transpose(1)
        write_desc(1, glast).start()
        write_desc(0, glast - 1).wait()
        write_desc(1, glast).wait()

    return k


def kernel(x, table):
    B, H = x.shape
    V, D = table.shape
    info = plsc.get_sparse_core_info()
    NW = info.num_cores * info.num_subcores
    xg = x.T.reshape((B // _G) * H, _G).astype(jnp.int32)
    out5 = _build(B, H, V, D, NW)(xg, table)
    return jnp.transpose(out5, (2, 4, 0, 1, 3)).reshape(B, H, D)


# speedup vs baseline: 4.7469x; 1.2006x over previous
import functools

import jax
import jax.numpy as jnp
from jax import lax
from jax.experimental import pallas as pl
from jax.experimental.pallas import tpu as pltpu
from jax.experimental.pallas import tpu_sc as plsc

_L = 16
_G = 128
_XB = 32


def _build(B, H, V, D, NW):
    n_groups = (B // _G) * H
    gpw = n_groups // NW
    n_pairs = gpw // 2
    DT, DS = D // 8, 8

    mesh = plsc.VectorSubcoreMesh(core_axis_name="c", subcore_axis_name="s")

    @functools.partial(
        pl.kernel,
        mesh=mesh,
        compiler_params=pltpu.CompilerParams(
            use_tc_tiling_on_sc=False, needs_layout_passes=False
        ),
        out_type=jax.ShapeDtypeStruct((H, DT, B // _G, DS, _G), jnp.float32),
        scratch_types=[
            pltpu.VMEM((_XB, _G), jnp.int32),
            pltpu.VMEM((2, _G), jnp.int32),
            pltpu.VMEM((2, _G, D), jnp.float32),
            pltpu.VMEM((2, DT, DS, _G), jnp.float32),
            pltpu.SemaphoreType.DMA,
            pltpu.SemaphoreType.DMA,
            pltpu.SemaphoreType.DMA,
            pltpu.SemaphoreType.DMA,
        ],
    )
    def k(xg_hbm, table_hbm, out_hbm, xbuf, idxbuf, rowbuf, tbuf, g0, g1, w0, w1):
        nc = lax.axis_size("c")
        wid = lax.axis_index("s") * nc + lax.axis_index("c")
        gbase = wid * gpw
        gsem = (g0, g1)
        wsem = (w0, w1)
        lane = jax.lax.iota(jnp.int32, _L)
        dsv = lax.rem(lane, 8)
        dtv = [dv * 2 + lax.div(lane, 8) for dv in range(D // _L)]

        def make_idx(slot, r):
            for v in range(_G // _L):
                s = pl.ds(v * _L, _L)
                idxbuf[slot, s] = -xbuf[r, s] - 1

        def fire_gather(slot):
            pltpu.async_copy(
                table_hbm.at[idxbuf.at[slot]], rowbuf.at[slot], gsem[slot]
            )

        def wait_gather(slot):
            pltpu.make_async_copy(
                table_hbm.at[idxbuf.at[slot]], rowbuf.at[slot], gsem[slot]
            ).wait()

        def transpose(slot):
            def tb(bo, carry):
                for bi in range(_L):
                    b = bo * _L + bi
                    blv = jnp.full((_L,), b, dtype=jnp.int32)
                    for dv in range(D // _L):
                        vec = rowbuf[slot, b, pl.ds(dv * _L, _L)]
                        plsc.store_scatter(
                            tbuf.at[slot], [dtv[dv], dsv, blv], vec
                        )
                return carry

            lax.fori_loop(0, _G // _L, tb, 0)

        def write_desc(slot, g):
            h = lax.div(g, B // _G)
            bt = lax.rem(g, B // _G)
            return pltpu.make_async_copy(
                tbuf.at[slot], out_hbm.at[h, :, bt], wsem[slot]
            )

        def body(p, carry):
            li0 = 2 * p
            gg0 = gbase + li0

            @pl.when(lax.rem(li0, _XB) == 0)
            def _load_x():
                pltpu.sync_copy(xg_hbm.at[pl.ds(gg0, _XB)], xbuf)

            r0 = lax.rem(li0, _XB)
            make_idx(0, r0)
            fire_gather(0)

            @pl.when(p >= 1)
            def _finish_prev_slot1():
                wait_gather(1)

                @pl.when(p >= 2)
                def _w1():
                    write_desc(1, gg0 - 3).wait()

                transpose(1)
                write_desc(1, gg0 - 1).start()

            make_idx(1, r0 + 1)
            fire_gather(1)

            wait_gather(0)

            @pl.when(p >= 1)
            def _w0():
                write_desc(0, gg0 - 2).wait()

            transpose(0)
            write_desc(0, gg0).start()
            return carry

        lax.fori_loop(0, n_pairs, body, 0)
        glast = gbase + gpw - 1
        wait_gather(1)
        write_desc(1, glast - 2).wait()
        transpose(1)
        write_desc(1, glast).start()
        write_desc(0, glast - 1).wait()
        write_desc(1, glast).wait()

    return k


def kernel(x, table):
    B, H = x.shape
    V, D = table.shape
    info = plsc.get_sparse_core_info()
    NW = info.num_cores * info.num_subcores
    xg = x.T.reshape((B // _G) * H, _G).astype(jnp.int32)
    out5 = _build(B, H, V, D, NW)(xg, table)
    return jnp.transpose(out5, (2, 4, 0, 1, 3)).reshape(B, H, D)


# speedup vs baseline: 10.1344x; 2.1350x over previous
import functools

import jax
import jax.numpy as jnp
from jax import lax
from jax.experimental import pallas as pl
from jax.experimental.pallas import tpu as pltpu
from jax.experimental.pallas import tpu_sc as plsc

_L = 16
_G = 128
_XB = 32


def _build(B, H, V, D, NW):
    n_groups = (B // _G) * H
    gpw = n_groups // NW
    n_pairs = gpw // 2
    DT, DS = D // 8, 8

    mesh = plsc.VectorSubcoreMesh(core_axis_name="c", subcore_axis_name="s")

    @functools.partial(
        pl.kernel,
        mesh=mesh,
        compiler_params=pltpu.CompilerParams(
            use_tc_tiling_on_sc=False, needs_layout_passes=False
        ),
        out_type=jax.ShapeDtypeStruct((H, DT, B // _G, DS, _G), jnp.float32),
        scratch_types=[
            pltpu.VMEM((_XB, _G), jnp.int32),
            pltpu.VMEM((2, _G), jnp.int32),
            pltpu.VMEM((2, _G, D), jnp.float32),
            pltpu.VMEM((2, DT, DS, _G + 1), jnp.float32),
            pltpu.SemaphoreType.DMA,
            pltpu.SemaphoreType.DMA,
            pltpu.SemaphoreType.DMA,
            pltpu.SemaphoreType.DMA,
        ],
    )
    def k(xg_hbm, table_hbm, out_hbm, xbuf, idxbuf, rowbuf, tbuf, g0, g1, w0, w1):
        nc = lax.axis_size("c")
        wid = lax.axis_index("s") * nc + lax.axis_index("c")
        gbase = wid * gpw
        gsem = (g0, g1)
        wsem = (w0, w1)
        lane = jax.lax.iota(jnp.int32, _L)
        dsv = lax.rem(lane, 8)
        dtv = [dv * 2 + lax.div(lane, 8) for dv in range(D // _L)]

        def make_idx(slot, r):
            for v in range(_G // _L):
                s = pl.ds(v * _L, _L)
                idxbuf[slot, s] = -xbuf[r, s] - 1

        def fire_gather(slot):
            pltpu.async_copy(
                table_hbm.at[idxbuf.at[slot]], rowbuf.at[slot], gsem[slot]
            )

        def wait_gather(slot):
            pltpu.make_async_copy(
                table_hbm.at[idxbuf.at[slot]], rowbuf.at[slot], gsem[slot]
            ).wait()

        def transpose(slot):
            def tb(bo, carry):
                for bi in range(_L):
                    b = bo * _L + bi
                    blv = jnp.full((_L,), b, dtype=jnp.int32)
                    for dv in range(D // _L):
                        vec = rowbuf[slot, b, pl.ds(dv * _L, _L)]
                        plsc.store_scatter(
                            tbuf.at[slot], [dtv[dv], dsv, blv], vec
                        )
                return carry

            lax.fori_loop(0, _G // _L, tb, 0)

        def write_desc(slot, g):
            h = lax.div(g, B // _G)
            bt = lax.rem(g, B // _G)
            return pltpu.make_async_copy(
                tbuf.at[slot, :, :, pl.ds(0, _G)], out_hbm.at[h, :, bt],
                wsem[slot],
            )

        def body(p, carry):
            li0 = 2 * p
            gg0 = gbase + li0

            @pl.when(lax.rem(li0, _XB) == 0)
            def _load_x():
                pltpu.sync_copy(xg_hbm.at[pl.ds(gg0, _XB)], xbuf)

            r0 = lax.rem(li0, _XB)
            make_idx(0, r0)
            fire_gather(0)

            @pl.when(p >= 1)
            def _finish_prev_slot1():
                wait_gather(1)

                @pl.when(p >= 2)
                def _w1():
                    write_desc(1, gg0 - 3).wait()

                transpose(1)
                write_desc(1, gg0 - 1).start()

            make_idx(1, r0 + 1)
            fire_gather(1)

            wait_gather(0)

            @pl.when(p >= 1)
            def _w0():
                write_desc(0, gg0 - 2).wait()

            transpose(0)
            write_desc(0, gg0).start()
            return carry

        lax.fori_loop(0, n_pairs, body, 0)
        glast = gbase + gpw - 1
        wait_gather(1)
        write_desc(1, glast - 2).wait()
        transpose(1)
        write_desc(1, glast).start()
        write_desc(0, glast - 1).wait()
        write_desc(1, glast).wait()

    return k


def kernel(x, table):
    B, H = x.shape
    V, D = table.shape
    info = plsc.get_sparse_core_info()
    NW = info.num_cores * info.num_subcores
    xg = x.T.reshape((B // _G) * H, _G).astype(jnp.int32)
    out5 = _build(B, H, V, D, NW)(xg, table)
    return jnp.transpose(out5, (2, 4, 0, 1, 3)).reshape(B, H, D)
